# baseline (speedup 1.0000x reference)
import jax
import jax.numpy as jnp
from jax import lax
from jax.experimental import pallas as pl
from jax.experimental.pallas import tpu as pltpu
from jax.experimental.pallas import tpu_sc as plsc

VOCAB_N = 100000
SEQ_N = 2048
DIM_N = 768
BATCH_N = 4

NUM_CORES = 2
NUM_SUBCORES = 16
NUM_WORKERS = NUM_CORES * NUM_SUBCORES
SEQ_PER_WORKER = SEQ_N // NUM_WORKERS
CHUNK = 32
SUBSPANS = SEQ_PER_WORKER // CHUNK
NUM_CHUNKS = SUBSPANS * BATCH_N
LANES = 16


def _sc_embed(idx_hbm, table_hbm, pos_hbm, out_sum_hbm, out_raw_hbm,
              idx0_v, idx1_v, idx2_v, idx3_v,
              raw0_v, raw1_v, sum0_v, sum1_v, pos_v,
              gsem0, gsem1, rsem0, rsem1, ssem0, ssem1, psem, isem):
    wid = lax.axis_index("s") * NUM_CORES + lax.axis_index("c")
    seq_base = wid * SEQ_PER_WORKER
    idx_v = (idx0_v, idx1_v, idx2_v, idx3_v)
    raw_v = (raw0_v, raw1_v)
    sum_v = (sum0_v, sum1_v)
    gsem = (gsem0, gsem1)
    rsem = (rsem0, rsem1)
    ssem = (ssem0, ssem1)

    def chunk_offsets(i):
        ss, b = divmod(i, BATCH_N)
        return seq_base + ss * CHUNK, b, ss

    idx_loads = [
        pltpu.async_copy(
            idx_hbm.at[b, pl.ds(seq_base, SEQ_PER_WORKER)],
            idx_v[b], isem)
        for b in range(BATCH_N)
    ]
    pos_load = pltpu.async_copy(pos_hbm.at[pl.ds(seq_base, CHUNK)],
                                pos_v, psem)
    for ld in idx_loads:
        ld.wait()
    gathers = [pltpu.async_copy(
        table_hbm.at[idx_v[0].at[pl.ds(0, CHUNK)]], raw_v[0], gsem[0]),
        None]
    raw_writes = [None, None]
    sum_writes = [None, None]

    for i in range(NUM_CHUNKS):
        p = i % 2
        row_off, b, ss = chunk_offsets(i)
        gathers[p].wait()
        raw_writes[p] = pltpu.async_copy(
            raw_v[p], out_raw_hbm.at[b, pl.ds(row_off, CHUNK)], rsem[p])

        if i + 1 < NUM_CHUNKS:
            q = 1 - p
            if raw_writes[q] is not None:
                raw_writes[q].wait()
                raw_writes[q] = None
            _, nb, nss = chunk_offsets(i + 1)
            gathers[q] = pltpu.async_copy(
                table_hbm.at[idx_v[nb].at[pl.ds(nss * CHUNK, CHUNK)]],
                raw_v[q], gsem[q])

        if i % BATCH_N == 0:
            pos_load.wait()
        if sum_writes[p] is not None:
            sum_writes[p].wait()
            sum_writes[p] = None

        rv, sv = raw_v[p], sum_v[p]

        @plsc.parallel_loop(0, CHUNK, unroll=4)
        def add_row(r, rv=rv, sv=sv):
            for j in range(DIM_N // LANES):
                sl = pl.ds(j * LANES, LANES)
                sv[r, sl] = pos_v[r, sl] + rv[r, sl]

        sum_writes[p] = pltpu.async_copy(
            sv, out_sum_hbm.at[b, pl.ds(row_off, CHUNK)], ssem[p])

        if i % BATCH_N == BATCH_N - 1 and i + 1 < NUM_CHUNKS:
            pos_load = pltpu.async_copy(
                pos_hbm.at[pl.ds(seq_base + (ss + 1) * CHUNK, CHUNK)],
                pos_v, psem)

    for ws in raw_writes + sum_writes:
        if ws is not None:
            ws.wait()


_sc_call = pl.kernel(
    _sc_embed,
    out_type=(
        jax.ShapeDtypeStruct((BATCH_N, SEQ_N, DIM_N), jnp.float32),
        jax.ShapeDtypeStruct((BATCH_N, SEQ_N, DIM_N), jnp.float32),
    ),
    mesh=plsc.VectorSubcoreMesh(core_axis_name="c", subcore_axis_name="s"),
    scratch_types=[
        pltpu.VMEM((SEQ_PER_WORKER,), jnp.int32),
        pltpu.VMEM((SEQ_PER_WORKER,), jnp.int32),
        pltpu.VMEM((SEQ_PER_WORKER,), jnp.int32),
        pltpu.VMEM((SEQ_PER_WORKER,), jnp.int32),
        pltpu.VMEM((CHUNK, DIM_N), jnp.float32),
        pltpu.VMEM((CHUNK, DIM_N), jnp.float32),
        pltpu.VMEM((CHUNK, DIM_N), jnp.float32),
        pltpu.VMEM((CHUNK, DIM_N), jnp.float32),
        pltpu.VMEM((CHUNK, DIM_N), jnp.float32),
        pltpu.SemaphoreType.DMA,
        pltpu.SemaphoreType.DMA,
        pltpu.SemaphoreType.DMA,
        pltpu.SemaphoreType.DMA,
        pltpu.SemaphoreType.DMA,
        pltpu.SemaphoreType.DMA,
        pltpu.SemaphoreType.DMA,
        pltpu.SemaphoreType.DMA,
    ],
)


@jax.jit
def kernel(inputs, embed_table, pos_embs):
    idx = inputs.astype(jnp.int32)
    out_sum, out_raw = _sc_call(idx, embed_table, pos_embs)
    return out_sum, out_raw


# speedup vs baseline: 1.1039x; 1.1039x over previous
import jax
import jax.numpy as jnp
from jax import lax
from jax.experimental import pallas as pl
from jax.experimental.pallas import tpu as pltpu
from jax.experimental.pallas import tpu_sc as plsc

VOCAB_N = 100000
SEQ_N = 2048
DIM_N = 768
BATCH_N = 4

NUM_CORES = 2
NUM_SUBCORES = 16
NUM_WORKERS = NUM_CORES * NUM_SUBCORES
SEQ_PER_WORKER = SEQ_N // NUM_WORKERS
CHUNK = 32
SUBSPANS = SEQ_PER_WORKER // CHUNK
NUM_CHUNKS = SUBSPANS * BATCH_N
LANES = 16


def _sc_embed(idx_hbm, table_hbm, pos_hbm, out_sum_hbm, out_raw_hbm,
              idx0_v, idx1_v, idx2_v, idx3_v,
              raw0_v, raw1_v, sum0_v, sum1_v, pos_v,
              gsem0, gsem1, rsem0, rsem1, ssem0, ssem1, psem, isem):
    wid = lax.axis_index("s") * NUM_CORES + lax.axis_index("c")
    seq_base = wid * SEQ_PER_WORKER
    idx_v = (idx0_v, idx1_v, idx2_v, idx3_v)
    raw_v = (raw0_v, raw1_v)
    sum_v = (sum0_v, sum1_v)
    gsem = (gsem0, gsem1)
    rsem = (rsem0, rsem1)
    ssem = (ssem0, ssem1)

    def chunk_offsets(i):
        ss, b = divmod(i, BATCH_N)
        return seq_base + ss * CHUNK, b, ss

    idx_loads = [
        pltpu.async_copy(
            idx_hbm.at[b, pl.ds(seq_base, SEQ_PER_WORKER)],
            idx_v[b], isem)
        for b in range(BATCH_N)
    ]
    pos_load = pltpu.async_copy(pos_hbm.at[pl.ds(seq_base, CHUNK)],
                                pos_v, psem)
    for ld in idx_loads:
        ld.wait()
    gathers = [pltpu.async_copy(
        table_hbm.at[idx_v[0].at[pl.ds(0, CHUNK)]], raw_v[0], gsem[0]),
        None]
    raw_writes = [None, None]
    sum_writes = [None, None]

    for i in range(NUM_CHUNKS):
        p = i % 2
        row_off, b, ss = chunk_offsets(i)
        gathers[p].wait()
        raw_writes[p] = pltpu.async_copy(
            raw_v[p], out_raw_hbm.at[b, pl.ds(row_off, CHUNK)], rsem[p])

        if i + 1 < NUM_CHUNKS:
            q = 1 - p
            if raw_writes[q] is not None:
                raw_writes[q].wait()
                raw_writes[q] = None
            _, nb, nss = chunk_offsets(i + 1)
            gathers[q] = pltpu.async_copy(
                table_hbm.at[idx_v[nb].at[pl.ds(nss * CHUNK, CHUNK)]],
                raw_v[q], gsem[q])

        if i % BATCH_N == 0:
            pos_load.wait()
        if sum_writes[p] is not None:
            sum_writes[p].wait()
            sum_writes[p] = None

        rv, sv = raw_v[p], sum_v[p]

        @plsc.parallel_loop(0, CHUNK, unroll=1)
        def add_row(r, rv=rv, sv=sv):
            for j in range(DIM_N // LANES):
                sl = pl.ds(j * LANES, LANES)
                sv[r, sl] = pos_v[r, sl] + rv[r, sl]

        sum_writes[p] = pltpu.async_copy(
            sv, out_sum_hbm.at[b, pl.ds(row_off, CHUNK)], ssem[p])

        if i % BATCH_N == BATCH_N - 1 and i + 1 < NUM_CHUNKS:
            pos_load = pltpu.async_copy(
                pos_hbm.at[pl.ds(seq_base + (ss + 1) * CHUNK, CHUNK)],
                pos_v, psem)

    for ws in raw_writes + sum_writes:
        if ws is not None:
            ws.wait()


_sc_call = pl.kernel(
    _sc_embed,
    out_type=(
        jax.ShapeDtypeStruct((BATCH_N, SEQ_N, DIM_N), jnp.float32),
        jax.ShapeDtypeStruct((BATCH_N, SEQ_N, DIM_N), jnp.float32),
    ),
    mesh=plsc.VectorSubcoreMesh(core_axis_name="c", subcore_axis_name="s"),
    scratch_types=[
        pltpu.VMEM((SEQ_PER_WORKER,), jnp.int32),
        pltpu.VMEM((SEQ_PER_WORKER,), jnp.int32),
        pltpu.VMEM((SEQ_PER_WORKER,), jnp.int32),
        pltpu.VMEM((SEQ_PER_WORKER,), jnp.int32),
        pltpu.VMEM((CHUNK, DIM_N), jnp.float32),
        pltpu.VMEM((CHUNK, DIM_N), jnp.float32),
        pltpu.VMEM((CHUNK, DIM_N), jnp.float32),
        pltpu.VMEM((CHUNK, DIM_N), jnp.float32),
        pltpu.VMEM((CHUNK, DIM_N), jnp.float32),
        pltpu.SemaphoreType.DMA,
        pltpu.SemaphoreType.DMA,
        pltpu.SemaphoreType.DMA,
        pltpu.SemaphoreType.DMA,
        pltpu.SemaphoreType.DMA,
        pltpu.SemaphoreType.DMA,
        pltpu.SemaphoreType.DMA,
        pltpu.SemaphoreType.DMA,
    ],
)


@jax.jit
def kernel(inputs, embed_table, pos_embs):
    idx = inputs.astype(jnp.int32)
    out_sum, out_raw = _sc_call(idx, embed_table, pos_embs)
    return out_sum, out_raw
